# baseline (speedup 1.0000x reference)
import functools

import jax
import jax.numpy as jnp
from jax import lax
from jax.experimental import pallas as pl
from jax.experimental.pallas import tpu as pltpu
from jax.experimental.pallas import tpu_sc as plsc

NC = 2
NS = 16
NW = NC * NS
L = 16
CH = 80


def _proj_kernel(x_ref, wem_ref, w_ref, ap_ref, hem_ref, s_ref):
    x = x_ref[...]
    hem_ref[...] = jnp.dot(x, wem_ref[...], preferred_element_type=jnp.float32)
    wa = jnp.dot(w_ref[...], ap_ref[...], preferred_element_type=jnp.float32)
    s_ref[...] = jnp.dot(x, wa, preferred_element_type=jnp.float32)


def _combine_kernel(p_ref, d_ref, out_ref):
    num = p_ref[0] + p_ref[1]
    den = d_ref[0] + d_ref[1]
    out_ref[...] = num / (den + 1e-16)


def _make_edge_kernel(N, E, D):
    EPW = E // NW
    NCHUNK = EPW // CH
    NMAIN = NCHUNK - 1
    RPT = N // NS
    Q, R = RPT // CH, RPT % CH
    KG = D // L
    DZ = 1000

    mesh = plsc.VectorSubcoreMesh(core_axis_name="c", subcore_axis_name="s")

    @functools.partial(
        pl.kernel,
        out_type=[
            jax.ShapeDtypeStruct((NW, NCHUNK, CH), jnp.float32),
            jax.ShapeDtypeStruct((NC, N, D), jnp.float32),
            jax.ShapeDtypeStruct((NC, N), jnp.float32),
        ],
        mesh=mesh,
        scratch_types=[
            pltpu.VMEM((4, 2, CH), jnp.int32),
            pltpu.VMEM((2 * N,), jnp.float32),
            pltpu.VMEM((2, CH), jnp.float32),
            pltpu.VMEM((2, CH), jnp.float32),
            pltpu.VMEM((1008,), jnp.float32),
            pltpu.VMEM((2, CH, D), jnp.float32),
            pltpu.VMEM_SHARED((N, D), jnp.float32),
            pltpu.VMEM_SHARED((N,), jnp.float32),
            dict(
                g=[pltpu.SemaphoreType.DMA] * 2,
                i=[pltpu.SemaphoreType.DMA] * 4,
                e=[pltpu.SemaphoreType.DMA] * 2,
                s=[pltpu.SemaphoreType.DMA] * 2,
                d=[pltpu.SemaphoreType.DMA] * 2,
            ),
        ],
        compiler_params=pltpu.CompilerParams(
            needs_layout_passes=False, use_tc_tiling_on_sc=False),
    )
    def edge_kernel(sd_hbm, s_hbm, hem_hbm, ee_hbm, num_hbm, den_hbm,
                    sd_v, s_v, ee_v, w_v, zden_v, rows_v, acc_sh, den_sh,
                    sem):
        c = lax.axis_index("c")
        t = lax.axis_index("s")
        wid = t * NC + c

        pltpu.sync_copy(s_hbm, s_v)

        zeros16 = jnp.zeros((L,), jnp.float32)

        def zrow(r, _):
            for k in range(KG):
                rows_v[0, r, pl.ds(k * L, L)] = zeros16
            return 0
        lax.fori_loop(0, CH, zrow, 0)
        r0 = t * RPT
        for q in range(Q):
            pltpu.sync_copy(rows_v.at[0], acc_sh.at[pl.ds(r0 + q * CH, CH)])
        if R:
            pltpu.sync_copy(rows_v.at[0, pl.ds(0, R)],
                            acc_sh.at[pl.ds(r0 + Q * CH, R)])

        for k in range(1008 // L):
            zden_v[pl.ds(k * L, L)] = zeros16

        @pl.when(t < N // DZ)
        def _():
            pltpu.sync_copy(zden_v.at[pl.ds(0, DZ)],
                            den_sh.at[pl.ds(t * DZ, DZ)])
        plsc.subcore_barrier()

        def issue_gather(jj, slot, par):
            return pltpu.async_copy(
                hem_hbm.at[sd_v.at[slot, 1]], rows_v.at[par], sem["g"][par])

        def wait_gather(jj, slot, par):
            pltpu.make_async_copy(
                hem_hbm.at[sd_v.at[slot, 1]], rows_v.at[par],
                sem["g"][par]).wait()

        def issue_idx(jj, slot):
            pltpu.async_copy(sd_hbm.at[0, wid, jj], sd_v.at[slot, 0],
                             sem["i"][slot])
            pltpu.async_copy(sd_hbm.at[1, wid, jj], sd_v.at[slot, 1],
                             sem["i"][slot])

        def wait_idx(jj, slot):
            pltpu.make_async_copy(sd_hbm.at[0, wid, jj], sd_v.at[slot, 0],
                                  sem["i"][slot]).wait()
            pltpu.make_async_copy(sd_hbm.at[1, wid, jj], sd_v.at[slot, 1],
                                  sem["i"][slot]).wait()

        def issue_scatter(slot, par):
            pltpu.async_copy(rows_v.at[par], acc_sh.at[sd_v.at[slot, 0]],
                             sem["s"][par], add=True)
            pltpu.async_copy(w_v.at[par], den_sh.at[sd_v.at[slot, 0]],
                             sem["d"][par], add=True)

        def wait_scatter(slot, par):
            pltpu.make_async_copy(rows_v.at[par],
                                  acc_sh.at[sd_v.at[slot, 0]],
                                  sem["s"][par]).wait()
            pltpu.make_async_copy(w_v.at[par],
                                  den_sh.at[sd_v.at[slot, 0]],
                                  sem["d"][par]).wait()

        def scalar_pass(jj, slot, par):
            wvecs = []
            for g in range(CH // L):
                si = sd_v[slot, 0, pl.ds(g * L, L)]
                di = sd_v[slot, 1, pl.ds(g * L, L)]
                v1 = plsc.load_gather(s_v, [si * 2])
                v2 = plsc.load_gather(s_v, [di * 2 + 1])
                ee = v1 + v2
                ee_v[par, pl.ds(g * L, L)] = ee
                sig = 1.0 / (1.0 + jnp.exp(-ee))
                wv = jnp.exp(sig)
                w_v[par, pl.ds(g * L, L)] = wv
                wvecs.append(wv)
            pltpu.async_copy(ee_v.at[par], ee_hbm.at[wid, jj], sem["e"][par])
            return wvecs

        def wait_ee(jj, par):
            pltpu.make_async_copy(ee_v.at[par], ee_hbm.at[wid, jj],
                                  sem["e"][par]).wait()

        def scale(wvecs, par):
            del wvecs

            @plsc.parallel_loop(0, CH, unroll=16)
            def _(e):
                wgrp = w_v[par, pl.ds((e // L) * L, L)]
                wb = jnp.take_along_axis(
                    wgrp, jnp.broadcast_to(e % L, (L,)).astype(jnp.int32),
                    axis=0)
                for k in range(KG):
                    rows_v[par, e, pl.ds(k * L, L)] = (
                        rows_v[par, e, pl.ds(k * L, L)] * wb)

        pltpu.sync_copy(sd_hbm.at[0, wid, 0], sd_v.at[0, 0])
        pltpu.sync_copy(sd_hbm.at[1, wid, 0], sd_v.at[0, 1])
        issue_gather(0, 0, 0)
        issue_idx(1, 1)
        issue_idx(2, 2)

        def body(m, _):
            for u in range(4):
                jj = m * 4 + u
                par = u % 2
                slot = u

                @pl.when(jj > 0)
                def _():
                    wait_scatter((u - 1) % 4, 1 - par)

                wait_idx(jj + 1, (u + 1) % 4)
                issue_gather(jj + 1, (u + 1) % 4, 1 - par)

                @pl.when(jj + 3 < NCHUNK)
                def _():
                    issue_idx(jj + 3, (u + 3) % 4)

                @pl.when(jj >= 2)
                def _():
                    wait_ee(jj - 2, par)
                wvecs = scalar_pass(jj, slot, par)

                wait_gather(jj, slot, par)
                scale(wvecs, par)
                issue_scatter(slot, par)
            return 0

        lax.fori_loop(0, NMAIN // 4, body, 0)

        jl = NCHUNK - 1
        wait_scatter(3, 1)
        wait_ee(jl - 2, 0)
        wvecs = scalar_pass(jl, 0, 0)
        wait_gather(jl, 0, 0)
        scale(wvecs, 0)
        issue_scatter(0, 0)
        wait_scatter(0, 0)
        wait_ee(jl - 1, 1)
        wait_ee(jl, 0)

        plsc.subcore_barrier()
        pltpu.sync_copy(acc_sh.at[pl.ds(r0, RPT)],
                        num_hbm.at[c, pl.ds(r0, RPT)])

        @pl.when(t < N // DZ)
        def _():
            pltpu.sync_copy(den_sh.at[pl.ds(t * DZ, DZ)],
                            den_hbm.at[c, pl.ds(t * DZ, DZ)])

    return edge_kernel


def kernel(input, edge_index, W, a, W_em):
    N, D_IN = input.shape
    D = W_em.shape[1]
    E = edge_index.shape[1]
    EPW = E // NW
    NCHUNK = EPW // CH

    a_pair = jnp.stack([a[:D, 0], a[D:, 0]], axis=1)

    BA = 1000
    hem, s = pl.pallas_call(
        _proj_kernel,
        grid=(N // BA,),
        in_specs=[
            pl.BlockSpec((BA, D_IN), lambda i: (i, 0)),
            pl.BlockSpec((D_IN, D), lambda i: (0, 0)),
            pl.BlockSpec((D_IN, D), lambda i: (0, 0)),
            pl.BlockSpec((D_IN, 2), lambda i: (0, 0)),
        ],
        out_specs=[
            pl.BlockSpec((BA, D), lambda i: (i, 0)),
            pl.BlockSpec((BA, 2), lambda i: (i, 0)),
        ],
        out_shape=[
            jax.ShapeDtypeStruct((N, D), jnp.float32),
            jax.ShapeDtypeStruct((N, 2), jnp.float32),
        ],
    )(input, W_em, W, a_pair)

    sd = edge_index.reshape(2, NW, NCHUNK, CH)
    ee, num, den = _make_edge_kernel(N, E, D)(sd, s.reshape(2 * N), hem)

    BC = 1000
    h_prime = pl.pallas_call(
        _combine_kernel,
        grid=(N // BC,),
        in_specs=[
            pl.BlockSpec((2, BC, D), lambda i: (0, i, 0)),
            pl.BlockSpec((2, BC, 1), lambda i: (0, i, 0)),
        ],
        out_specs=pl.BlockSpec((BC, D), lambda i: (i, 0)),
        out_shape=jax.ShapeDtypeStruct((N, D), jnp.float32),
    )(num, den.reshape(NC, N, 1))

    edge_e = ee.reshape(E, 1)
    return (h_prime, edge_e)


# speedup vs baseline: 1.0436x; 1.0436x over previous
import functools

import jax
import jax.numpy as jnp
from jax import lax
from jax.experimental import pallas as pl
from jax.experimental.pallas import tpu as pltpu
from jax.experimental.pallas import tpu_sc as plsc

NC = 2
NS = 16
NW = NC * NS
L = 16
CH = 80


def _proj_kernel(x_ref, wem_ref, w_ref, ap_ref, hem_ref, s_ref):
    x = x_ref[...]
    hem_ref[...] = jnp.dot(x, wem_ref[...], preferred_element_type=jnp.float32)
    wa = jnp.dot(w_ref[...], ap_ref[...], preferred_element_type=jnp.float32)
    s_ref[...] = jnp.dot(x, wa, preferred_element_type=jnp.float32)


def _combine_kernel(p_ref, d_ref, out_ref):
    num = p_ref[0] + p_ref[1]
    den = d_ref[0] + d_ref[1]
    out_ref[...] = num / (den + 1e-16)


def _make_edge_kernel(N, E, D):
    EPW = E // NW
    NCHUNK = EPW // CH
    NMAIN = NCHUNK - 1
    RPT = N // NS
    Q, R = RPT // CH, RPT % CH
    KG = D // L
    DZ = 1000

    mesh = plsc.VectorSubcoreMesh(core_axis_name="c", subcore_axis_name="s")

    @functools.partial(
        pl.kernel,
        out_type=[
            jax.ShapeDtypeStruct((NW, NCHUNK, CH), jnp.float32),
            jax.ShapeDtypeStruct((NC, N, D), jnp.float32),
            jax.ShapeDtypeStruct((NC, N), jnp.float32),
        ],
        mesh=mesh,
        scratch_types=[
            pltpu.VMEM((4, 2, CH), jnp.int32),
            pltpu.VMEM((2 * N,), jnp.float32),
            pltpu.VMEM((2, CH), jnp.float32),
            pltpu.VMEM((2, CH), jnp.float32),
            pltpu.VMEM((1008,), jnp.float32),
            pltpu.VMEM((2, CH, D), jnp.float32),
            pltpu.VMEM_SHARED((N, D), jnp.float32),
            pltpu.VMEM_SHARED((N,), jnp.float32),
            dict(
                g=[pltpu.SemaphoreType.DMA] * 2,
                i=[pltpu.SemaphoreType.DMA] * 4,
                e=[pltpu.SemaphoreType.DMA] * 2,
                s=[pltpu.SemaphoreType.DMA] * 2,
                d=[pltpu.SemaphoreType.DMA] * 2,
            ),
        ],
        compiler_params=pltpu.CompilerParams(
            needs_layout_passes=False, use_tc_tiling_on_sc=False),
    )
    def edge_kernel(sd_hbm, s_hbm, hem_hbm, ee_hbm, num_hbm, den_hbm,
                    sd_v, s_v, ee_v, w_v, zden_v, rows_v, acc_sh, den_sh,
                    sem):
        c = lax.axis_index("c")
        t = lax.axis_index("s")
        wid = t * NC + c

        pltpu.sync_copy(s_hbm, s_v)

        zeros16 = jnp.zeros((L,), jnp.float32)

        def zrow(r, _):
            for k in range(KG):
                rows_v[0, r, pl.ds(k * L, L)] = zeros16
            return 0
        lax.fori_loop(0, CH, zrow, 0)
        r0 = t * RPT
        for q in range(Q):
            pltpu.sync_copy(rows_v.at[0], acc_sh.at[pl.ds(r0 + q * CH, CH)])
        if R:
            pltpu.sync_copy(rows_v.at[0, pl.ds(0, R)],
                            acc_sh.at[pl.ds(r0 + Q * CH, R)])

        for k in range(1008 // L):
            zden_v[pl.ds(k * L, L)] = zeros16

        @pl.when(t < N // DZ)
        def _():
            pltpu.sync_copy(zden_v.at[pl.ds(0, DZ)],
                            den_sh.at[pl.ds(t * DZ, DZ)])
        plsc.subcore_barrier()

        def issue_gather(jj, slot, par):
            return pltpu.async_copy(
                hem_hbm.at[sd_v.at[slot, 1]], rows_v.at[par], sem["g"][par])

        def wait_gather(jj, slot, par):
            pltpu.make_async_copy(
                hem_hbm.at[sd_v.at[slot, 1]], rows_v.at[par],
                sem["g"][par]).wait()

        def issue_idx(jj, slot):
            pltpu.async_copy(sd_hbm.at[0, wid, jj], sd_v.at[slot, 0],
                             sem["i"][slot])
            pltpu.async_copy(sd_hbm.at[1, wid, jj], sd_v.at[slot, 1],
                             sem["i"][slot])

        def wait_idx(jj, slot):
            pltpu.make_async_copy(sd_hbm.at[0, wid, jj], sd_v.at[slot, 0],
                                  sem["i"][slot]).wait()
            pltpu.make_async_copy(sd_hbm.at[1, wid, jj], sd_v.at[slot, 1],
                                  sem["i"][slot]).wait()

        def issue_scatter(slot, par):
            pltpu.async_copy(rows_v.at[par], acc_sh.at[sd_v.at[slot, 0]],
                             sem["s"][par], add=True)
            pltpu.async_copy(w_v.at[par], den_sh.at[sd_v.at[slot, 0]],
                             sem["d"][par], add=True)

        def wait_scatter(slot, par):
            pltpu.make_async_copy(rows_v.at[par],
                                  acc_sh.at[sd_v.at[slot, 0]],
                                  sem["s"][par]).wait()
            pltpu.make_async_copy(w_v.at[par],
                                  den_sh.at[sd_v.at[slot, 0]],
                                  sem["d"][par]).wait()

        def scalar_pass(jj, slot, par):
            wvecs = []
            for g in range(CH // L):
                si = sd_v[slot, 0, pl.ds(g * L, L)]
                di = sd_v[slot, 1, pl.ds(g * L, L)]
                v1 = plsc.load_gather(s_v, [si * 2])
                v2 = plsc.load_gather(s_v, [di * 2 + 1])
                ee = v1 + v2
                ee_v[par, pl.ds(g * L, L)] = ee
                sig = 1.0 / (1.0 + jnp.exp(-ee))
                wv = jnp.exp(sig)
                w_v[par, pl.ds(g * L, L)] = wv
                wvecs.append(wv)
            pltpu.async_copy(ee_v.at[par], ee_hbm.at[wid, jj], sem["e"][par])
            return wvecs

        def wait_ee(jj, par):
            pltpu.make_async_copy(ee_v.at[par], ee_hbm.at[wid, jj],
                                  sem["e"][par]).wait()

        def scale(wvecs, par):
            del wvecs

            @plsc.parallel_loop(0, CH, unroll=4)
            def _(e):
                wgrp = w_v[par, pl.ds((e // L) * L, L)]
                wb = jnp.take_along_axis(
                    wgrp, jnp.broadcast_to(e % L, (L,)).astype(jnp.int32),
                    axis=0)
                for k in range(KG):
                    rows_v[par, e, pl.ds(k * L, L)] = (
                        rows_v[par, e, pl.ds(k * L, L)] * wb)

        pltpu.sync_copy(sd_hbm.at[0, wid, 0], sd_v.at[0, 0])
        pltpu.sync_copy(sd_hbm.at[1, wid, 0], sd_v.at[0, 1])
        issue_gather(0, 0, 0)
        issue_idx(1, 1)
        issue_idx(2, 2)

        def body(m, _):
            for u in range(4):
                jj = m * 4 + u
                par = u % 2
                slot = u

                @pl.when(jj > 0)
                def _():
                    wait_scatter((u - 1) % 4, 1 - par)

                wait_idx(jj + 1, (u + 1) % 4)
                issue_gather(jj + 1, (u + 1) % 4, 1 - par)

                @pl.when(jj + 3 < NCHUNK)
                def _():
                    issue_idx(jj + 3, (u + 3) % 4)

                @pl.when(jj >= 2)
                def _():
                    wait_ee(jj - 2, par)
                wvecs = scalar_pass(jj, slot, par)

                wait_gather(jj, slot, par)
                scale(wvecs, par)
                issue_scatter(slot, par)
            return 0

        lax.fori_loop(0, NMAIN // 4, body, 0)

        jl = NCHUNK - 1
        wait_scatter(3, 1)
        wait_ee(jl - 2, 0)
        wvecs = scalar_pass(jl, 0, 0)
        wait_gather(jl, 0, 0)
        scale(wvecs, 0)
        issue_scatter(0, 0)
        wait_scatter(0, 0)
        wait_ee(jl - 1, 1)
        wait_ee(jl, 0)

        plsc.subcore_barrier()
        pltpu.sync_copy(acc_sh.at[pl.ds(r0, RPT)],
                        num_hbm.at[c, pl.ds(r0, RPT)])

        @pl.when(t < N // DZ)
        def _():
            pltpu.sync_copy(den_sh.at[pl.ds(t * DZ, DZ)],
                            den_hbm.at[c, pl.ds(t * DZ, DZ)])

    return edge_kernel


def kernel(input, edge_index, W, a, W_em):
    N, D_IN = input.shape
    D = W_em.shape[1]
    E = edge_index.shape[1]
    EPW = E // NW
    NCHUNK = EPW // CH

    a_pair = jnp.stack([a[:D, 0], a[D:, 0]], axis=1)

    BA = 1000
    hem, s = pl.pallas_call(
        _proj_kernel,
        grid=(N // BA,),
        in_specs=[
            pl.BlockSpec((BA, D_IN), lambda i: (i, 0)),
            pl.BlockSpec((D_IN, D), lambda i: (0, 0)),
            pl.BlockSpec((D_IN, D), lambda i: (0, 0)),
            pl.BlockSpec((D_IN, 2), lambda i: (0, 0)),
        ],
        out_specs=[
            pl.BlockSpec((BA, D), lambda i: (i, 0)),
            pl.BlockSpec((BA, 2), lambda i: (i, 0)),
        ],
        out_shape=[
            jax.ShapeDtypeStruct((N, D), jnp.float32),
            jax.ShapeDtypeStruct((N, 2), jnp.float32),
        ],
    )(input, W_em, W, a_pair)

    sd = edge_index.reshape(2, NW, NCHUNK, CH)
    ee, num, den = _make_edge_kernel(N, E, D)(sd, s.reshape(2 * N), hem)

    BC = 1000
    h_prime = pl.pallas_call(
        _combine_kernel,
        grid=(N // BC,),
        in_specs=[
            pl.BlockSpec((2, BC, D), lambda i: (0, i, 0)),
            pl.BlockSpec((2, BC, 1), lambda i: (0, i, 0)),
        ],
        out_specs=pl.BlockSpec((BC, D), lambda i: (i, 0)),
        out_shape=jax.ShapeDtypeStruct((N, D), jnp.float32),
    )(num, den.reshape(NC, N, 1))

    edge_e = ee.reshape(E, 1)
    return (h_prime, edge_e)


# speedup vs baseline: 1.0444x; 1.0007x over previous
import functools

import jax
import jax.numpy as jnp
from jax import lax
from jax.experimental import pallas as pl
from jax.experimental.pallas import tpu as pltpu
from jax.experimental.pallas import tpu_sc as plsc

NC = 2
NS = 16
NW = NC * NS
L = 16
CH = 80


def _proj_kernel(x_ref, wem_ref, w_ref, ap_ref, hem_ref, s_ref):
    x = x_ref[...]
    hem_ref[...] = jnp.dot(x, wem_ref[...], preferred_element_type=jnp.float32)
    wa = jnp.dot(w_ref[...], ap_ref[...], preferred_element_type=jnp.float32)
    s_ref[...] = jnp.dot(x, wa, preferred_element_type=jnp.float32)


def _combine_kernel(p_ref, d_ref, out_ref):
    num = p_ref[0] + p_ref[1]
    den = d_ref[0] + d_ref[1]
    out_ref[...] = num / (den + 1e-16)


def _make_edge_kernel(N, E, D):
    EPW = E // NW
    NCHUNK = EPW // CH
    NMAIN = NCHUNK - 1
    RPT = N // NS
    Q, R = RPT // CH, RPT % CH
    KG = D // L
    DZ = 1000

    mesh = plsc.VectorSubcoreMesh(core_axis_name="c", subcore_axis_name="s")

    @functools.partial(
        pl.kernel,
        out_type=[
            jax.ShapeDtypeStruct((NW, NCHUNK, CH), jnp.float32),
            jax.ShapeDtypeStruct((NC, N, D), jnp.float32),
            jax.ShapeDtypeStruct((NC, N), jnp.float32),
        ],
        mesh=mesh,
        scratch_types=[
            pltpu.VMEM((4, 2, CH), jnp.int32),
            pltpu.VMEM((2 * N,), jnp.float32),
            pltpu.VMEM((2, 4, CH), jnp.float32),
            pltpu.VMEM((2, CH), jnp.float32),
            pltpu.VMEM((1008,), jnp.float32),
            pltpu.VMEM((2, CH, D), jnp.float32),
            pltpu.VMEM_SHARED((N, D), jnp.float32),
            pltpu.VMEM_SHARED((N,), jnp.float32),
            dict(
                g=[pltpu.SemaphoreType.DMA] * 2,
                i=[pltpu.SemaphoreType.DMA] * 4,
                e=[pltpu.SemaphoreType.DMA] * 1,
                s=[pltpu.SemaphoreType.DMA] * 2,
                d=[pltpu.SemaphoreType.DMA] * 2,
            ),
        ],
        compiler_params=pltpu.CompilerParams(
            needs_layout_passes=False, use_tc_tiling_on_sc=False),
    )
    def edge_kernel(sd_hbm, s_hbm, hem_hbm, ee_hbm, num_hbm, den_hbm,
                    sd_v, s_v, ee_v, w_v, zden_v, rows_v, acc_sh, den_sh,
                    sem):
        c = lax.axis_index("c")
        t = lax.axis_index("s")
        wid = t * NC + c

        pltpu.sync_copy(s_hbm, s_v)

        zeros16 = jnp.zeros((L,), jnp.float32)

        def zrow(r, _):
            for k in range(KG):
                rows_v[0, r, pl.ds(k * L, L)] = zeros16
            return 0
        lax.fori_loop(0, CH, zrow, 0)
        r0 = t * RPT
        for q in range(Q):
            pltpu.sync_copy(rows_v.at[0], acc_sh.at[pl.ds(r0 + q * CH, CH)])
        if R:
            pltpu.sync_copy(rows_v.at[0, pl.ds(0, R)],
                            acc_sh.at[pl.ds(r0 + Q * CH, R)])

        for k in range(1008 // L):
            zden_v[pl.ds(k * L, L)] = zeros16

        @pl.when(t < N // DZ)
        def _():
            pltpu.sync_copy(zden_v.at[pl.ds(0, DZ)],
                            den_sh.at[pl.ds(t * DZ, DZ)])
        plsc.subcore_barrier()

        def issue_gather(jj, slot, par):
            return pltpu.async_copy(
                hem_hbm.at[sd_v.at[slot, 1]], rows_v.at[par], sem["g"][par])

        def wait_gather(jj, slot, par):
            pltpu.make_async_copy(
                hem_hbm.at[sd_v.at[slot, 1]], rows_v.at[par],
                sem["g"][par]).wait()

        def issue_idx(jj, slot):
            pltpu.async_copy(sd_hbm.at[0, wid, jj], sd_v.at[slot, 0],
                             sem["i"][slot])
            pltpu.async_copy(sd_hbm.at[1, wid, jj], sd_v.at[slot, 1],
                             sem["i"][slot])

        def wait_idx(jj, slot):
            pltpu.make_async_copy(sd_hbm.at[0, wid, jj], sd_v.at[slot, 0],
                                  sem["i"][slot]).wait()
            pltpu.make_async_copy(sd_hbm.at[1, wid, jj], sd_v.at[slot, 1],
                                  sem["i"][slot]).wait()

        def issue_scatter(slot, par):
            pltpu.async_copy(rows_v.at[par], acc_sh.at[sd_v.at[slot, 0]],
                             sem["s"][par], add=True)
            pltpu.async_copy(w_v.at[par], den_sh.at[sd_v.at[slot, 0]],
                             sem["d"][par], add=True)

        def wait_scatter(slot, par):
            pltpu.make_async_copy(rows_v.at[par],
                                  acc_sh.at[sd_v.at[slot, 0]],
                                  sem["s"][par]).wait()
            pltpu.make_async_copy(w_v.at[par],
                                  den_sh.at[sd_v.at[slot, 0]],
                                  sem["d"][par]).wait()

        def scalar_pass(pm, u, slot, par):
            for g in range(CH // L):
                si = sd_v[slot, 0, pl.ds(g * L, L)]
                di = sd_v[slot, 1, pl.ds(g * L, L)]
                v1 = plsc.load_gather(s_v, [si * 2])
                v2 = plsc.load_gather(s_v, [di * 2 + 1])
                ee = v1 + v2
                ee_v[pm, u, pl.ds(g * L, L)] = ee
                sig = 1.0 / (1.0 + jnp.exp(-ee))
                wv = jnp.exp(sig)
                w_v[par, pl.ds(g * L, L)] = wv

        def scale(wvecs, par):
            del wvecs

            @plsc.parallel_loop(0, CH, unroll=4)
            def _(e):
                wgrp = w_v[par, pl.ds((e // L) * L, L)]
                wb = jnp.take_along_axis(
                    wgrp, jnp.broadcast_to(e % L, (L,)).astype(jnp.int32),
                    axis=0)
                for k in range(KG):
                    rows_v[par, e, pl.ds(k * L, L)] = (
                        rows_v[par, e, pl.ds(k * L, L)] * wb)

        pltpu.sync_copy(sd_hbm.at[0, wid, 0], sd_v.at[0, 0])
        pltpu.sync_copy(sd_hbm.at[1, wid, 0], sd_v.at[0, 1])
        issue_gather(0, 0, 0)
        issue_idx(1, 1)
        issue_idx(2, 2)

        def body(m, _):
            pm = m % 2
            for u in range(4):
                jj = m * 4 + u
                par = u % 2
                slot = u

                @pl.when(jj > 0)
                def _():
                    wait_scatter((u - 1) % 4, 1 - par)

                wait_idx(jj + 1, (u + 1) % 4)
                issue_gather(jj + 1, (u + 1) % 4, 1 - par)

                @pl.when(jj + 3 < NCHUNK)
                def _():
                    issue_idx(jj + 3, (u + 3) % 4)

                scalar_pass(pm, u, slot, par)

                wait_gather(jj, slot, par)
                scale(None, par)
                issue_scatter(slot, par)

            @pl.when(m > 0)
            def _():
                pltpu.make_async_copy(
                    ee_v.at[1 - pm], ee_hbm.at[wid, pl.ds((m - 1) * 4, 4)],
                    sem["e"][0]).wait()
            pltpu.async_copy(ee_v.at[pm], ee_hbm.at[wid, pl.ds(m * 4, 4)],
                             sem["e"][0])
            return 0

        lax.fori_loop(0, NMAIN // 4, body, 0)

        NB = NMAIN // 4
        jl = NCHUNK - 1
        wait_scatter(3, 1)
        scalar_pass(1, 0, 0, 0)
        wait_gather(jl, 0, 0)
        scale(None, 0)
        issue_scatter(0, 0)
        pltpu.make_async_copy(
            ee_v.at[(NB - 1) % 2], ee_hbm.at[wid, pl.ds((NB - 1) * 4, 4)],
            sem["e"][0]).wait()
        pltpu.async_copy(ee_v.at[1, 0], ee_hbm.at[wid, jl], sem["e"][0])
        pltpu.make_async_copy(ee_v.at[1, 0], ee_hbm.at[wid, jl],
                              sem["e"][0]).wait()
        wait_scatter(0, 0)

        plsc.subcore_barrier()
        pltpu.sync_copy(acc_sh.at[pl.ds(r0, RPT)],
                        num_hbm.at[c, pl.ds(r0, RPT)])

        @pl.when(t < N // DZ)
        def _():
            pltpu.sync_copy(den_sh.at[pl.ds(t * DZ, DZ)],
                            den_hbm.at[c, pl.ds(t * DZ, DZ)])

    return edge_kernel


def kernel(input, edge_index, W, a, W_em):
    N, D_IN = input.shape
    D = W_em.shape[1]
    E = edge_index.shape[1]
    EPW = E // NW
    NCHUNK = EPW // CH

    a_pair = jnp.stack([a[:D, 0], a[D:, 0]], axis=1)

    BA = 1000
    hem, s = pl.pallas_call(
        _proj_kernel,
        grid=(N // BA,),
        in_specs=[
            pl.BlockSpec((BA, D_IN), lambda i: (i, 0)),
            pl.BlockSpec((D_IN, D), lambda i: (0, 0)),
            pl.BlockSpec((D_IN, D), lambda i: (0, 0)),
            pl.BlockSpec((D_IN, 2), lambda i: (0, 0)),
        ],
        out_specs=[
            pl.BlockSpec((BA, D), lambda i: (i, 0)),
            pl.BlockSpec((BA, 2), lambda i: (i, 0)),
        ],
        out_shape=[
            jax.ShapeDtypeStruct((N, D), jnp.float32),
            jax.ShapeDtypeStruct((N, 2), jnp.float32),
        ],
    )(input, W_em, W, a_pair)

    sd = edge_index.reshape(2, NW, NCHUNK, CH)
    ee, num, den = _make_edge_kernel(N, E, D)(sd, s.reshape(2 * N), hem)

    BC = 1000
    h_prime = pl.pallas_call(
        _combine_kernel,
        grid=(N // BC,),
        in_specs=[
            pl.BlockSpec((2, BC, D), lambda i: (0, i, 0)),
            pl.BlockSpec((2, BC, 1), lambda i: (0, i, 0)),
        ],
        out_specs=pl.BlockSpec((BC, D), lambda i: (i, 0)),
        out_shape=jax.ShapeDtypeStruct((N, D), jnp.float32),
    )(num, den.reshape(NC, N, 1))

    edge_e = ee.reshape(E, 1)
    return (h_prime, edge_e)


# speedup vs baseline: 1.0525x; 1.0078x over previous
import functools

import jax
import jax.numpy as jnp
from jax import lax
from jax.experimental import pallas as pl
from jax.experimental.pallas import tpu as pltpu
from jax.experimental.pallas import tpu_sc as plsc

NC = 2
NS = 16
NW = NC * NS
L = 16
CH = 80


def _proj_kernel(x_ref, wem_ref, w_ref, ap_ref, hem_ref, s_ref):
    x = x_ref[...]
    hem_ref[...] = jnp.dot(x, wem_ref[...], preferred_element_type=jnp.float32)
    wa = jnp.dot(w_ref[...], ap_ref[...], preferred_element_type=jnp.float32)
    s_ref[...] = jnp.dot(x, wa, preferred_element_type=jnp.float32)


def _combine_kernel(p_ref, d_ref, out_ref):
    num = p_ref[0] + p_ref[1]
    den = d_ref[0] + d_ref[1]
    out_ref[...] = num / (den + 1e-16)


def _make_edge_kernel(N, E, D):
    EPW = E // NW
    NCHUNK = EPW // CH
    NMAIN = NCHUNK - 1
    NB = NMAIN // 4
    RPT = N // NS
    Q, R = RPT // CH, RPT % CH
    KG = D // L
    DZ = 1000

    mesh = plsc.VectorSubcoreMesh(core_axis_name="c", subcore_axis_name="s")

    @functools.partial(
        pl.kernel,
        out_type=[
            jax.ShapeDtypeStruct((NW, NCHUNK, CH), jnp.float32),
            jax.ShapeDtypeStruct((NC, N, D), jnp.float32),
            jax.ShapeDtypeStruct((NC, N), jnp.float32),
        ],
        mesh=mesh,
        scratch_types=[
            pltpu.VMEM((8, 2, CH), jnp.int32),
            pltpu.VMEM((4, 2, CH), jnp.float32),
            pltpu.VMEM((2, 4, CH), jnp.float32),
            pltpu.VMEM((4, CH), jnp.float32),
            pltpu.VMEM((1008,), jnp.float32),
            pltpu.VMEM((4, CH, D), jnp.float32),
            pltpu.VMEM_SHARED((N, D), jnp.float32),
            pltpu.VMEM_SHARED((N,), jnp.float32),
            dict(
                g=[pltpu.SemaphoreType.DMA] * 4,
                i=[pltpu.SemaphoreType.DMA] * 8,
                e=[pltpu.SemaphoreType.DMA] * 1,
                s=[pltpu.SemaphoreType.DMA] * 4,
                d=[pltpu.SemaphoreType.DMA] * 4,
            ),
        ],
        compiler_params=pltpu.CompilerParams(
            needs_layout_passes=False, use_tc_tiling_on_sc=False),
    )
    def edge_kernel(sd_hbm, s1_hbm, s2_hbm, hem_hbm, ee_hbm, num_hbm, den_hbm,
                    sd_v, sg_v, ee_v, w_v, zden_v, rows_v, acc_sh, den_sh,
                    sem):
        c = lax.axis_index("c")
        t = lax.axis_index("s")
        wid = t * NC + c

        zeros16 = jnp.zeros((L,), jnp.float32)

        def issue_idx(jj, q):
            pltpu.async_copy(sd_hbm.at[0, wid, jj], sd_v.at[q, 0],
                             sem["i"][q])
            pltpu.async_copy(sd_hbm.at[1, wid, jj], sd_v.at[q, 1],
                             sem["i"][q])

        def wait_idx(jj, q):
            pltpu.make_async_copy(sd_hbm.at[0, wid, jj], sd_v.at[q, 0],
                                  sem["i"][q]).wait()
            pltpu.make_async_copy(sd_hbm.at[1, wid, jj], sd_v.at[q, 1],
                                  sem["i"][q]).wait()

        def issue_gather(q, r):
            pltpu.async_copy(hem_hbm.at[sd_v.at[q, 1]], rows_v.at[r],
                             sem["g"][r])
            pltpu.async_copy(s1_hbm.at[sd_v.at[q, 0]], sg_v.at[r, 0],
                             sem["g"][r])
            pltpu.async_copy(s2_hbm.at[sd_v.at[q, 1]], sg_v.at[r, 1],
                             sem["g"][r])

        def wait_gather(q, r):
            pltpu.make_async_copy(hem_hbm.at[sd_v.at[q, 1]], rows_v.at[r],
                                  sem["g"][r]).wait()
            pltpu.make_async_copy(s1_hbm.at[sd_v.at[q, 0]], sg_v.at[r, 0],
                                  sem["g"][r]).wait()
            pltpu.make_async_copy(s2_hbm.at[sd_v.at[q, 1]], sg_v.at[r, 1],
                                  sem["g"][r]).wait()

        def issue_scatter(q, r):
            pltpu.async_copy(rows_v.at[r], acc_sh.at[sd_v.at[q, 0]],
                             sem["s"][r], add=True)
            pltpu.async_copy(w_v.at[r], den_sh.at[sd_v.at[q, 0]],
                             sem["d"][r], add=True)

        def wait_scatter(q, r):
            pltpu.make_async_copy(rows_v.at[r],
                                  acc_sh.at[sd_v.at[q, 0]],
                                  sem["s"][r]).wait()
            pltpu.make_async_copy(w_v.at[r],
                                  den_sh.at[sd_v.at[q, 0]],
                                  sem["d"][r]).wait()

        def scalar_pass(pm, u, r):
            for g in range(CH // L):
                v1 = sg_v[r, 0, pl.ds(g * L, L)]
                v2 = sg_v[r, 1, pl.ds(g * L, L)]
                ee = v1 + v2
                ee_v[pm, u, pl.ds(g * L, L)] = ee
                sig = 1.0 / (1.0 + jnp.exp(-ee))
                w_v[r, pl.ds(g * L, L)] = jnp.exp(sig)

        def scale(r):
            @plsc.parallel_loop(0, CH, unroll=4)
            def _(e):
                wgrp = w_v[r, pl.ds((e // L) * L, L)]
                wb = jnp.take_along_axis(
                    wgrp, jnp.broadcast_to(e % L, (L,)).astype(jnp.int32),
                    axis=0)
                for k in range(KG):
                    rows_v[r, e, pl.ds(k * L, L)] = (
                        rows_v[r, e, pl.ds(k * L, L)] * wb)

        pltpu.sync_copy(sd_hbm.at[0, wid, 0], sd_v.at[0, 0])
        pltpu.sync_copy(sd_hbm.at[1, wid, 0], sd_v.at[0, 1])
        issue_idx(1, 1)
        issue_idx(2, 2)
        issue_idx(3, 3)

        def zrow(rr, _):
            for k in range(KG):
                rows_v[0, rr, pl.ds(k * L, L)] = zeros16
            return 0
        lax.fori_loop(0, CH, zrow, 0)
        r0 = t * RPT
        for q in range(Q):
            pltpu.sync_copy(rows_v.at[0], acc_sh.at[pl.ds(r0 + q * CH, CH)])
        if R:
            pltpu.sync_copy(rows_v.at[0, pl.ds(0, R)],
                            acc_sh.at[pl.ds(r0 + Q * CH, R)])

        for k in range(1008 // L):
            zden_v[pl.ds(k * L, L)] = zeros16

        @pl.when(t < N // DZ)
        def _():
            pltpu.sync_copy(zden_v.at[pl.ds(0, DZ)],
                            den_sh.at[pl.ds(t * DZ, DZ)])
        plsc.subcore_barrier()

        issue_gather(0, 0)

        def chunk_core(jj, v):
            r = v % 4
            q = v
            q1 = (v + 1) % 8
            r1 = (v + 1) % 4

            wait_idx(jj + 1, q1)
            issue_gather(q1, r1)

            wait_gather(q, r)
            scalar_pass(v // 4, v % 4, r)
            scale(r)
            issue_scatter(q, r)

        def body(m8, _):
            for v in range(8):
                jj = m8 * 8 + v

                if v == 0 or v == 4:
                    pm = v // 4

                    @pl.when(m8 > 0)
                    def _():
                        pltpu.make_async_copy(
                            ee_v.at[pm],
                            ee_hbm.at[wid, pl.ds((m8 - 1) * 8 + pm * 4, 4)],
                            sem["e"][0]).wait()

                @pl.when(jj >= 2)
                def _():
                    wait_scatter((v - 2) % 8, (v - 2) % 4)

                issue_idx(jj + 4, (v + 4) % 8)

                chunk_core(jj, v)

                if v == 3 or v == 7:
                    pm = v // 4
                    pltpu.async_copy(
                        ee_v.at[pm],
                        ee_hbm.at[wid, pl.ds(m8 * 8 + pm * 4, 4)],
                        sem["e"][0])
            return 0

        NB8 = (NCHUNK - 5) // 8
        lax.fori_loop(0, NB8, body, 0)

        t0 = NB8 * 8
        pltpu.make_async_copy(
            ee_v.at[0], ee_hbm.at[wid, pl.ds(t0 - 8, 4)], sem["e"][0]).wait()
        for jj in range(t0, t0 + 4):
            v = jj % 8
            wait_scatter((v - 2) % 8, (v - 2) % 4)
            if jj + 4 < NCHUNK:
                issue_idx(jj + 4, (v + 4) % 8)
            chunk_core(jj, v)
        pltpu.async_copy(ee_v.at[0], ee_hbm.at[wid, pl.ds(t0, 4)],
                         sem["e"][0])

        jl = NCHUNK - 1
        wait_scatter(2, 2)
        wait_scatter(3, 3)
        pltpu.make_async_copy(
            ee_v.at[1], ee_hbm.at[wid, pl.ds(t0 - 4, 4)], sem["e"][0]).wait()
        pltpu.make_async_copy(
            ee_v.at[0], ee_hbm.at[wid, pl.ds(t0, 4)], sem["e"][0]).wait()
        wait_gather(4, 0)
        scalar_pass(1, 0, 0)
        scale(0)
        issue_scatter(4, 0)
        pltpu.async_copy(ee_v.at[1, 0], ee_hbm.at[wid, jl], sem["e"][0])
        pltpu.make_async_copy(ee_v.at[1, 0], ee_hbm.at[wid, jl],
                              sem["e"][0]).wait()
        wait_scatter(4, 0)

        plsc.subcore_barrier()
        pltpu.sync_copy(acc_sh.at[pl.ds(r0, RPT)],
                        num_hbm.at[c, pl.ds(r0, RPT)])

        @pl.when(t < N // DZ)
        def _():
            pltpu.sync_copy(den_sh.at[pl.ds(t * DZ, DZ)],
                            den_hbm.at[c, pl.ds(t * DZ, DZ)])

    return edge_kernel


def kernel(input, edge_index, W, a, W_em):
    N, D_IN = input.shape
    D = W_em.shape[1]
    E = edge_index.shape[1]
    EPW = E // NW
    NCHUNK = EPW // CH

    a_pair = jnp.stack([a[:D, 0], a[D:, 0]], axis=1)

    BA = 1000
    hem, s = pl.pallas_call(
        _proj_kernel,
        grid=(N // BA,),
        in_specs=[
            pl.BlockSpec((BA, D_IN), lambda i: (i, 0)),
            pl.BlockSpec((D_IN, D), lambda i: (0, 0)),
            pl.BlockSpec((D_IN, D), lambda i: (0, 0)),
            pl.BlockSpec((D_IN, 2), lambda i: (0, 0)),
        ],
        out_specs=[
            pl.BlockSpec((BA, D), lambda i: (i, 0)),
            pl.BlockSpec((BA, 2), lambda i: (i, 0)),
        ],
        out_shape=[
            jax.ShapeDtypeStruct((N, D), jnp.float32),
            jax.ShapeDtypeStruct((N, 2), jnp.float32),
        ],
    )(input, W_em, W, a_pair)

    sd = edge_index.reshape(2, NW, NCHUNK, CH)
    ee, num, den = _make_edge_kernel(N, E, D)(
        sd, s[:, 0], s[:, 1], hem)

    BC = 1000
    h_prime = pl.pallas_call(
        _combine_kernel,
        grid=(N // BC,),
        in_specs=[
            pl.BlockSpec((2, BC, D), lambda i: (0, i, 0)),
            pl.BlockSpec((2, BC, 1), lambda i: (0, i, 0)),
        ],
        out_specs=pl.BlockSpec((BC, D), lambda i: (i, 0)),
        out_shape=jax.ShapeDtypeStruct((N, D), jnp.float32),
    )(num, den.reshape(NC, N, 1))

    edge_e = ee.reshape(E, 1)
    return (h_prime, edge_e)
